# baseline (speedup 1.0000x reference)
import jax
import jax.numpy as jnp
from jax.experimental import pallas as pl
from jax.experimental.pallas import tpu as pltpu

NUM_EXPERTS = 8
D = 768
D_FF = 1536
TOKENS = 2048
TILE_T = 512


def _gating_body(x_ref, wg1_ref, bg1_ref, wg2_ref, bg2_ref, wg3_ref, bg3_ref,
                 wd_ref, bd_ref, wfull_ref):
    def mm(a, b):
        return jnp.dot(a.astype(jnp.bfloat16), b.astype(jnp.bfloat16),
                       preferred_element_type=jnp.float32)

    x = x_ref[...]
    g = jax.nn.relu(mm(x, wg1_ref[...]) + bg1_ref[...])
    g = jax.nn.relu(mm(g, wg2_ref[...]) + bg2_ref[...])
    logits = mm(g, wg3_ref[...]) + bg3_ref[...]
    logits = logits + (mm(x, wd_ref[...]) + bd_ref[...]) * 0.1
    p = jax.nn.softmax(logits, axis=-1)
    lane = jax.lax.broadcasted_iota(jnp.int32, p.shape, 1)
    m1 = jnp.max(p, axis=-1, keepdims=True)
    i1 = jnp.min(jnp.where(p == m1, lane, NUM_EXPERTS), axis=-1, keepdims=True)
    pm = jnp.where(lane == i1, -jnp.inf, p)
    m2 = jnp.max(pm, axis=-1, keepdims=True)
    i2 = jnp.min(jnp.where(pm == m2, lane, NUM_EXPERTS), axis=-1, keepdims=True)
    e2 = jnp.exp(m2 - m1)
    s1 = 1.0 / (1.0 + e2)
    s2 = e2 / (1.0 + e2)
    s1 = jnp.where(s1 > 0.01, s1, 0.0)
    s2 = jnp.where(s2 > 0.01, s2, 0.0)
    wfull_ref[...] = (jnp.where(lane == i1, s1, 0.0)
                      + jnp.where(lane == i2, s2, 0.0))


def _expert_body(wfull_ref, x_ref, w1_ref, b1_ref, w2_ref, b2_ref, out_ref):
    hp = jax.lax.Precision.HIGHEST
    e = pl.program_id(1)
    x = x_ref[...]
    h = jax.nn.relu(jnp.dot(x, w1_ref[0], precision=hp) + b1_ref[0, 0])
    o = jnp.dot(h, w2_ref[0], precision=hp) + b2_ref[0, 0]
    conf = jax.nn.sigmoid(jnp.mean(o, axis=-1))
    lane = jax.lax.broadcasted_iota(jnp.int32, wfull_ref.shape, 1)
    w = jnp.sum(jnp.where(lane == e, wfull_ref[...], 0.0), axis=-1)

    @pl.when(e == 0)
    def _():
        out_ref[...] = jnp.zeros_like(out_ref)

    out_ref[...] += o * (w * conf)[:, None]


def kernel(x, Wg1, bg1, Wg2, bg2, Wg3, bg3, Wd, bd, W1, b1, W2, b2):
    wfull = pl.pallas_call(
        _gating_body,
        out_shape=jax.ShapeDtypeStruct((TOKENS, NUM_EXPERTS), jnp.float32),
    )(x, Wg1, bg1, Wg2, bg2, Wg3, bg3, Wd, bd)

    nt = TOKENS // TILE_T
    out = pl.pallas_call(
        _expert_body,
        grid=(nt, NUM_EXPERTS),
        in_specs=[
            pl.BlockSpec((TILE_T, NUM_EXPERTS), lambda t, e: (t, 0)),
            pl.BlockSpec((TILE_T, D), lambda t, e: (t, 0)),
            pl.BlockSpec((1, D, D_FF), lambda t, e: (e, 0, 0)),
            pl.BlockSpec((1, 1, D_FF), lambda t, e: (e, 0, 0)),
            pl.BlockSpec((1, D_FF, D), lambda t, e: (e, 0, 0)),
            pl.BlockSpec((1, 1, D), lambda t, e: (e, 0, 0)),
        ],
        out_specs=pl.BlockSpec((TILE_T, D), lambda t, e: (t, 0)),
        out_shape=jax.ShapeDtypeStruct((TOKENS, D), jnp.float32),
    )(wfull, x, W1, b1.reshape(NUM_EXPERTS, 1, D_FF), W2,
      b2.reshape(NUM_EXPERTS, 1, D))
    return out


# speedup vs baseline: 3.5056x; 3.5056x over previous
import jax
import jax.numpy as jnp
from jax.experimental import pallas as pl
from jax.experimental.pallas import tpu as pltpu

NUM_EXPERTS = 8
D = 768
D_FF = 1536
TOKENS = 2048
TILE_T = 1024


def _gating_body(x_ref, wg1_ref, bg1_ref, wg2_ref, bg2_ref, wg3_ref, bg3_ref,
                 wd_ref, bd_ref, wfull_ref):
    def mm(a, b):
        return jnp.dot(a.astype(jnp.bfloat16), b.astype(jnp.bfloat16),
                       preferred_element_type=jnp.float32)

    x = x_ref[...]
    g = jax.nn.relu(mm(x, wg1_ref[...]) + bg1_ref[...])
    g = jax.nn.relu(mm(g, wg2_ref[...]) + bg2_ref[...])
    logits = mm(g, wg3_ref[...]) + bg3_ref[...]
    logits = logits + (mm(x, wd_ref[...]) + bd_ref[...]) * 0.1
    p = jax.nn.softmax(logits, axis=-1)
    lane = jax.lax.broadcasted_iota(jnp.int32, p.shape, 1)
    m1 = jnp.max(p, axis=-1, keepdims=True)
    i1 = jnp.min(jnp.where(p == m1, lane, NUM_EXPERTS), axis=-1, keepdims=True)
    pm = jnp.where(lane == i1, -jnp.inf, p)
    m2 = jnp.max(pm, axis=-1, keepdims=True)
    i2 = jnp.min(jnp.where(pm == m2, lane, NUM_EXPERTS), axis=-1, keepdims=True)
    e2 = jnp.exp(m2 - m1)
    s1 = 1.0 / (1.0 + e2)
    s2 = e2 / (1.0 + e2)
    s1 = jnp.where(s1 > 0.01, s1, 0.0)
    s2 = jnp.where(s2 > 0.01, s2, 0.0)
    wfull_ref[...] = (jnp.where(lane == i1, s1, 0.0)
                      + jnp.where(lane == i2, s2, 0.0))


def _expert_body(wfull_ref, x_ref, w1_ref, b1_ref, w2_ref, b2_ref, out_ref):
    e = pl.program_id(1)
    x = x_ref[...].astype(jnp.bfloat16)
    h = jax.nn.relu(
        jnp.dot(x, w1_ref[0], preferred_element_type=jnp.float32)
        + b1_ref[0, 0])
    o = jnp.dot(h.astype(jnp.bfloat16), w2_ref[0],
                preferred_element_type=jnp.float32) + b2_ref[0, 0]
    conf = jax.nn.sigmoid(jnp.mean(o, axis=-1))
    lane = jax.lax.broadcasted_iota(jnp.int32, wfull_ref.shape, 1)
    w = jnp.sum(jnp.where(lane == e, wfull_ref[...], 0.0), axis=-1)

    @pl.when(e == 0)
    def _():
        out_ref[...] = jnp.zeros_like(out_ref)

    out_ref[...] += o * (w * conf)[:, None]


def kernel(x, Wg1, bg1, Wg2, bg2, Wg3, bg3, Wd, bd, W1, b1, W2, b2):
    wfull = pl.pallas_call(
        _gating_body,
        out_shape=jax.ShapeDtypeStruct((TOKENS, NUM_EXPERTS), jnp.float32),
    )(x, Wg1, bg1, Wg2, bg2, Wg3, bg3, Wd, bd)

    nt = TOKENS // TILE_T
    out = pl.pallas_call(
        _expert_body,
        grid=(nt, NUM_EXPERTS),
        in_specs=[
            pl.BlockSpec((TILE_T, NUM_EXPERTS), lambda t, e: (t, 0)),
            pl.BlockSpec((TILE_T, D), lambda t, e: (t, 0)),
            pl.BlockSpec((1, D, D_FF), lambda t, e: (e, 0, 0)),
            pl.BlockSpec((1, 1, D_FF), lambda t, e: (e, 0, 0)),
            pl.BlockSpec((1, D_FF, D), lambda t, e: (e, 0, 0)),
            pl.BlockSpec((1, 1, D), lambda t, e: (e, 0, 0)),
        ],
        out_specs=pl.BlockSpec((TILE_T, D), lambda t, e: (t, 0)),
        out_shape=jax.ShapeDtypeStruct((TOKENS, D), jnp.float32),
    )(wfull, x, W1.astype(jnp.bfloat16), b1.reshape(NUM_EXPERTS, 1, D_FF),
      W2.astype(jnp.bfloat16), b2.reshape(NUM_EXPERTS, 1, D))
    return out
